# Initial kernel scaffold; baseline (speedup 1.0000x reference)
import jax
import jax.numpy as jnp
from jax.experimental import pallas as pl


def kernel(input, embed):
    raise NotImplementedError("write your pallas kernel here")


# speedup vs baseline: 1.7392x; 1.7392x over previous
import functools

import jax
import jax.numpy as jnp
from jax.experimental import pallas as pl
from jax.experimental.pallas import tpu as pltpu

_DIM = 64
_N_EMBED = 1024
_TOKENS = 16384
_BLK = 1024
_NUM_BLOCKS = _TOKENS // _BLK


def _vq_body(x_ref, e_ref, q_ref, ind_ref, diff_ref, ppl_ref, cnt_ref, dsum_ref):
    i = pl.program_id(0)

    @pl.when(i == 0)
    def _init():
        cnt_ref[...] = jnp.zeros_like(cnt_ref)
        dsum_ref[0, 0] = 0.0

    x = x_ref[...]
    e = e_ref[...]
    scores = jax.lax.dot_general(
        x, e, (((1,), (0,)), ((), ())), preferred_element_type=jnp.float32)
    x_sq = jnp.sum(x * x, axis=1, keepdims=True)
    e_sq = jnp.sum(e * e, axis=0, keepdims=True)
    neg_dist = -(x_sq - 2.0 * scores + e_sq)

    mx = jnp.max(neg_dist, axis=1, keepdims=True)
    iota = jax.lax.broadcasted_iota(jnp.int32, (_BLK, _N_EMBED), 1)
    ind = jnp.min(jnp.where(neg_dist == mx, iota, jnp.int32(1 << 30)), axis=1)

    onehot = (iota == ind[:, None]).astype(jnp.float32)
    q = jax.lax.dot_general(
        onehot, e, (((1,), (1,)), ((), ())), preferred_element_type=jnp.float32)

    q_ref[...] = x + (q - x)
    ind_ref[...] = ind

    cnt_ref[...] += jnp.sum(onehot, axis=0)
    dsum_ref[0, 0] += jnp.sum((q - x) ** 2)

    @pl.when(i == _NUM_BLOCKS - 1)
    def _fin():
        diff_ref[...] = jnp.reshape(dsum_ref[0, 0] / float(_TOKENS * _DIM), (1, 1))
        p = cnt_ref[...] / float(_TOKENS)
        ent = jnp.sum(p * jnp.log(jnp.clip(p, 1e-7, None)), keepdims=True)
        ppl_ref[...] = jnp.exp(-ent).reshape(1, 1)


@functools.partial(jax.jit, static_argnames=())
def kernel(input, embed):
    flat = input.reshape(-1, _DIM)
    q, ind, diff, ppl = pl.pallas_call(
        _vq_body,
        grid=(_NUM_BLOCKS,),
        in_specs=[
            pl.BlockSpec((_BLK, _DIM), lambda i: (i, 0)),
            pl.BlockSpec((_DIM, _N_EMBED), lambda i: (0, 0)),
        ],
        out_specs=[
            pl.BlockSpec((_BLK, _DIM), lambda i: (i, 0)),
            pl.BlockSpec((_BLK,), lambda i: (i,)),
            pl.BlockSpec((1, 1), lambda i: (0, 0)),
            pl.BlockSpec((1, 1), lambda i: (0, 0)),
        ],
        out_shape=[
            jax.ShapeDtypeStruct((_TOKENS, _DIM), jnp.float32),
            jax.ShapeDtypeStruct((_TOKENS,), jnp.int32),
            jax.ShapeDtypeStruct((1, 1), jnp.float32),
            jax.ShapeDtypeStruct((1, 1), jnp.float32),
        ],
        scratch_shapes=[
            pltpu.VMEM((_N_EMBED,), jnp.float32),
            pltpu.SMEM((1, 1), jnp.float32),
        ],
    )(flat, embed)
    quantize_st = q.reshape(input.shape)
    embed_ind = ind.reshape(input.shape[:-1])
    return quantize_st, diff[0, 0], embed_ind, ppl[0, 0]


# speedup vs baseline: 1.8952x; 1.0897x over previous
import functools

import jax
import jax.numpy as jnp
from jax.experimental import pallas as pl
from jax.experimental.pallas import tpu as pltpu

_DIM = 64
_N_EMBED = 1024
_TOKENS = 16384
_BLK = 2048
_NUM_BLOCKS = _TOKENS // _BLK


def _vq_body(x_ref, e_ref, q_ref, ind_ref, diff_ref, ppl_ref, cnt_ref, dsum_ref):
    i = pl.program_id(0)

    @pl.when(i == 0)
    def _init():
        cnt_ref[...] = jnp.zeros_like(cnt_ref)
        dsum_ref[0, 0] = 0.0

    x = x_ref[...]
    e = e_ref[...]
    scores = jax.lax.dot_general(
        x, e, (((1,), (0,)), ((), ())), preferred_element_type=jnp.float32)
    x_sq = jnp.sum(x * x, axis=1, keepdims=True)
    e_sq = jnp.sum(e * e, axis=0, keepdims=True)
    neg_dist = -(x_sq - 2.0 * scores + e_sq)

    ind = jnp.argmax(neg_dist, axis=1).astype(jnp.int32)
    iota = jax.lax.broadcasted_iota(jnp.int32, (_BLK, _N_EMBED), 1)
    onehot = (iota == ind[:, None]).astype(jnp.float32)
    q = jax.lax.dot_general(
        onehot, e, (((1,), (1,)), ((), ())), preferred_element_type=jnp.float32)

    q_ref[...] = x + (q - x)
    ind_ref[...] = ind

    cnt_ref[...] += jnp.sum(onehot, axis=0)
    dsum_ref[0, 0] += jnp.sum((q - x) ** 2)

    @pl.when(i == _NUM_BLOCKS - 1)
    def _fin():
        diff_ref[...] = jnp.reshape(dsum_ref[0, 0] / float(_TOKENS * _DIM), (1, 1))
        p = cnt_ref[...] / float(_TOKENS)
        ent = jnp.sum(p * jnp.log(jnp.clip(p, 1e-7, None)), keepdims=True)
        ppl_ref[...] = jnp.exp(-ent).reshape(1, 1)


@functools.partial(jax.jit, static_argnames=())
def kernel(input, embed):
    flat = input.reshape(-1, _DIM)
    q, ind, diff, ppl = pl.pallas_call(
        _vq_body,
        grid=(_NUM_BLOCKS,),
        in_specs=[
            pl.BlockSpec((_BLK, _DIM), lambda i: (i, 0)),
            pl.BlockSpec((_DIM, _N_EMBED), lambda i: (0, 0)),
        ],
        out_specs=[
            pl.BlockSpec((_BLK, _DIM), lambda i: (i, 0)),
            pl.BlockSpec((_BLK,), lambda i: (i,)),
            pl.BlockSpec((1, 1), lambda i: (0, 0)),
            pl.BlockSpec((1, 1), lambda i: (0, 0)),
        ],
        out_shape=[
            jax.ShapeDtypeStruct((_TOKENS, _DIM), jnp.float32),
            jax.ShapeDtypeStruct((_TOKENS,), jnp.int32),
            jax.ShapeDtypeStruct((1, 1), jnp.float32),
            jax.ShapeDtypeStruct((1, 1), jnp.float32),
        ],
        scratch_shapes=[
            pltpu.VMEM((_N_EMBED,), jnp.float32),
            pltpu.SMEM((1, 1), jnp.float32),
        ],
    )(flat, embed)
    quantize_st = q.reshape(input.shape)
    embed_ind = ind.reshape(input.shape[:-1])
    return quantize_st, diff[0, 0], embed_ind, ppl[0, 0]
